# baseline (speedup 1.0000x reference)
import functools

import jax
import jax.numpy as jnp
from jax import lax
from jax.experimental import pallas as pl
from jax.experimental.pallas import tpu as pltpu
from jax.experimental.pallas import tpu_sc as plsc

LANES = 16
NUM_CORES = 2
NUM_SUBCORES = 16
NUM_WORKERS = NUM_CORES * NUM_SUBCORES
ROW_CHUNK = 16
NBUF = 5


def _make_emb(batch, seq, vocab, d):
    pos_per_w = seq // NUM_WORKERS
    chunks = pos_per_w // ROW_CHUNK
    rounds = chunks * batch
    slices_per_row = d // LANES

    mesh = plsc.VectorSubcoreMesh(core_axis_name="c", subcore_axis_name="s")

    @functools.partial(
        pl.kernel,
        mesh=mesh,
        out_type=jax.ShapeDtypeStruct((batch * seq, d), jnp.float32),
        scratch_types=[
            pltpu.VMEM((batch * pos_per_w,), jnp.int32),
            pltpu.VMEM((ROW_CHUNK, d), jnp.float32),
            pltpu.VMEM((ROW_CHUNK, d), jnp.float32),
        ]
        + [pltpu.VMEM((ROW_CHUNK, d), jnp.float32) for _ in range(NBUF)]
        + [pltpu.SemaphoreType.DMA for _ in range(2 * NBUF + 3)],
    )
    def emb(x_hbm, tok_hbm, pos_hbm, out_hbm, idx_v, *refs):
        pbuf = refs[:2]
        rows = refs[2:2 + NBUF]
        gsem = refs[2 + NBUF:2 + 2 * NBUF]
        wsem = refs[2 + 2 * NBUF:2 + 3 * NBUF]
        psem = refs[2 + 3 * NBUF:2 + 3 * NBUF + 2]
        isem = refs[2 + 3 * NBUF + 2]

        wid = lax.axis_index("s") * NUM_CORES + lax.axis_index("c")
        p0 = wid * pos_per_w

        idx_cps = [
            pltpu.async_copy(
                x_hbm.at[pl.ds(b * seq + p0, pos_per_w)],
                idx_v.at[pl.ds(b * pos_per_w, pos_per_w)],
                isem,
            )
            for b in range(batch)
        ]

        def start_pos(c):
            return pltpu.async_copy(
                pos_hbm.at[pl.ds(p0 + c * ROW_CHUNK, ROW_CHUNK)],
                pbuf[c % 2], psem[c % 2],
            )

        def start_gather(r):
            c, b = divmod(r, batch)
            idx = idx_v.at[pl.ds(b * pos_per_w + c * ROW_CHUNK, ROW_CHUNK)]
            return pltpu.async_copy(tok_hbm.at[idx], rows[r % NBUF], gsem[r % NBUF])

        ahead = NBUF - 2
        pos_cp = [None, None]
        pos_cp[0] = start_pos(0)
        for cp in idx_cps:
            cp.wait()
        gcp = [None] * rounds
        wcp = [None] * rounds
        for r in range(ahead):
            gcp[r] = start_gather(r)

        for r in range(rounds):
            c, b = divmod(r, batch)
            if r + ahead < rounds:
                if r + ahead - NBUF >= 0:
                    for cp in wcp[r + ahead - NBUF]:
                        cp.wait()
                gcp[r + ahead] = start_gather(r + ahead)
            if b == 0:
                pos_cp[c % 2].wait()
                if c + 1 < chunks:
                    pos_cp[(c + 1) % 2] = start_pos(c + 1)
            gcp[r].wait()

            def add_row(h, carry, _buf=r % NBUF, _p=c % 2):
                i = h // 2
                j0 = (h % 2) * (slices_per_row // 2)
                for j in range(slices_per_row // 2):
                    plsc.addupdate(
                        rows[_buf].at[i, pl.ds((j0 + j) * LANES, LANES)],
                        pbuf[_p][i, pl.ds((j0 + j) * LANES, LANES)],
                    )
                return carry

            half = ROW_CHUNK // 2
            out_row0 = b * seq + p0 + c * ROW_CHUNK
            lax.fori_loop(0, ROW_CHUNK, add_row, 0)
            wcp[r] = [
                pltpu.async_copy(
                    rows[r % NBUF].at[pl.ds(0, half)],
                    out_hbm.at[pl.ds(out_row0, half)],
                    wsem[r % NBUF],
                )
            ]
            lax.fori_loop(ROW_CHUNK, 2 * ROW_CHUNK, add_row, 0)
            wcp[r].append(
                pltpu.async_copy(
                    rows[r % NBUF].at[pl.ds(half, half)],
                    out_hbm.at[pl.ds(out_row0 + half, half)],
                    wsem[r % NBUF],
                )
            )
        for r in range(rounds - NBUF, rounds):
            for cp in wcp[r]:
                cp.wait()

    return emb


def kernel(x, token_table, pos_table):
    batch, seq = x.shape
    vocab, d = token_table.shape
    xf = x.reshape(batch * seq).astype(jnp.int32)
    emb = _make_emb(batch, seq, vocab, d)
    out = emb(xf, token_table, pos_table)
    return out.reshape(batch, seq, d)


# speedup vs baseline: 1.0804x; 1.0804x over previous
import functools

import jax
import jax.numpy as jnp
from jax import lax
from jax.experimental import pallas as pl
from jax.experimental.pallas import tpu as pltpu
from jax.experimental.pallas import tpu_sc as plsc

LANES = 16
NUM_CORES = 2
NUM_SUBCORES = 16
NUM_WORKERS = NUM_CORES * NUM_SUBCORES
ROW_CHUNK = 16
NBUF = 5


def _make_emb(batch, seq, vocab, d):
    pos_per_w = seq // NUM_WORKERS
    chunks = pos_per_w // ROW_CHUNK
    rounds = chunks * batch
    slices_per_row = d // LANES

    mesh = plsc.VectorSubcoreMesh(core_axis_name="c", subcore_axis_name="s")

    @functools.partial(
        pl.kernel,
        mesh=mesh,
        out_type=jax.ShapeDtypeStruct((batch * seq, d), jnp.float32),
        scratch_types=[
            pltpu.VMEM((batch * pos_per_w,), jnp.int32),
            pltpu.VMEM((ROW_CHUNK, d), jnp.float32),
            pltpu.VMEM((ROW_CHUNK, d), jnp.float32),
        ]
        + [pltpu.VMEM((ROW_CHUNK, d), jnp.float32) for _ in range(NBUF)]
        + [pltpu.SemaphoreType.DMA for _ in range(2 * NBUF + 3)],
    )
    def emb(x_hbm, tok_hbm, pos_hbm, out_hbm, idx_v, *refs):
        pbuf = refs[:2]
        rows = refs[2:2 + NBUF]
        gsem = refs[2 + NBUF:2 + 2 * NBUF]
        wsem = refs[2 + 2 * NBUF:2 + 3 * NBUF]
        psem = refs[2 + 3 * NBUF:2 + 3 * NBUF + 2]
        isem = refs[2 + 3 * NBUF + 2]

        wid = lax.axis_index("s") * NUM_CORES + lax.axis_index("c")
        p0 = wid * pos_per_w

        idx_cps = [
            pltpu.async_copy(
                x_hbm.at[pl.ds(b * seq + p0, pos_per_w)],
                idx_v.at[pl.ds(b * pos_per_w, pos_per_w)],
                isem,
            )
            for b in range(batch)
        ]

        def start_pos(c):
            return pltpu.async_copy(
                pos_hbm.at[pl.ds(p0 + c * ROW_CHUNK, ROW_CHUNK)],
                pbuf[c % 2], psem[c % 2],
            )

        def start_gather(r):
            c, b = divmod(r, batch)
            idx = idx_v.at[pl.ds(b * pos_per_w + c * ROW_CHUNK, ROW_CHUNK)]
            return pltpu.async_copy(tok_hbm.at[idx], rows[r % NBUF], gsem[r % NBUF])

        ahead = NBUF - 2
        pos_cp = [None, None]
        pos_cp[0] = start_pos(0)
        for cp in idx_cps:
            cp.wait()
        gcp = [None] * rounds
        wcp = [None] * rounds
        for r in range(ahead):
            gcp[r] = start_gather(r)

        for r in range(rounds):
            c, b = divmod(r, batch)
            if r + ahead < rounds:
                if r + ahead - NBUF >= 0:
                    for cp in wcp[r + ahead - NBUF]:
                        cp.wait()
                gcp[r + ahead] = start_gather(r + ahead)
            if b == 0:
                pos_cp[c % 2].wait()
                if c + 1 < chunks:
                    pos_cp[(c + 1) % 2] = start_pos(c + 1)
            gcp[r].wait()

            @plsc.parallel_loop(0, ROW_CHUNK)
            def add_row(i, _buf=r % NBUF, _p=c % 2):
                for j in range(slices_per_row):
                    plsc.addupdate(
                        rows[_buf].at[i, pl.ds(j * LANES, LANES)],
                        pbuf[_p][i, pl.ds(j * LANES, LANES)],
                    )
            wcp[r] = [
                pltpu.async_copy(
                    rows[r % NBUF],
                    out_hbm.at[pl.ds(b * seq + p0 + c * ROW_CHUNK, ROW_CHUNK)],
                    wsem[r % NBUF],
                )
            ]
        for r in range(rounds - NBUF, rounds):
            for cp in wcp[r]:
                cp.wait()

    return emb


def kernel(x, token_table, pos_table):
    batch, seq = x.shape
    vocab, d = token_table.shape
    xf = x.reshape(batch * seq).astype(jnp.int32)
    emb = _make_emb(batch, seq, vocab, d)
    out = emb(xf, token_table, pos_table)
    return out.reshape(batch, seq, d)


# speedup vs baseline: 1.1855x; 1.0972x over previous
import functools

import jax
import jax.numpy as jnp
from jax import lax
from jax.experimental import pallas as pl
from jax.experimental.pallas import tpu as pltpu
from jax.experimental.pallas import tpu_sc as plsc

LANES = 16
NUM_CORES = 2
NUM_SUBCORES = 16
NUM_WORKERS = NUM_CORES * NUM_SUBCORES
ROW_CHUNK = 16
NBUF = 5


def _make_emb(batch, seq, vocab, d):
    pos_per_w = seq // NUM_WORKERS
    chunks = pos_per_w // ROW_CHUNK
    rounds = chunks * batch
    slices_per_row = d // LANES

    mesh = plsc.VectorSubcoreMesh(core_axis_name="c", subcore_axis_name="s")

    @functools.partial(
        pl.kernel,
        mesh=mesh,
        out_type=jax.ShapeDtypeStruct((batch * seq, d), jnp.float32),
        scratch_types=[
            pltpu.VMEM((batch * pos_per_w,), jnp.int32),
            pltpu.VMEM((ROW_CHUNK, d), jnp.float32),
            pltpu.VMEM((ROW_CHUNK, d), jnp.float32),
        ]
        + [pltpu.VMEM((ROW_CHUNK, d), jnp.float32) for _ in range(NBUF)]
        + [pltpu.SemaphoreType.DMA for _ in range(2 * NBUF + 3)],
    )
    def emb(x_hbm, tok_hbm, pos_hbm, out_hbm, idx_v, *refs):
        pbuf = refs[:2]
        rows = refs[2:2 + NBUF]
        gsem = refs[2 + NBUF:2 + 2 * NBUF]
        wsem = refs[2 + 2 * NBUF:2 + 3 * NBUF]
        psem = refs[2 + 3 * NBUF:2 + 3 * NBUF + 2]
        isem = refs[2 + 3 * NBUF + 2]

        wid = lax.axis_index("s") * NUM_CORES + lax.axis_index("c")
        p0 = wid * pos_per_w

        idx_cps = [
            pltpu.async_copy(
                x_hbm.at[pl.ds(b * seq + p0, pos_per_w)],
                idx_v.at[pl.ds(b * pos_per_w, pos_per_w)],
                isem,
            )
            for b in range(batch)
        ]

        def start_pos(c):
            return pltpu.async_copy(
                pos_hbm.at[pl.ds(p0 + c * ROW_CHUNK, ROW_CHUNK)],
                pbuf[c % 2], psem[c % 2],
            )

        def start_gather(r):
            c, b = divmod(r, batch)
            idx = idx_v.at[pl.ds(b * pos_per_w + c * ROW_CHUNK, ROW_CHUNK)]
            return pltpu.async_copy(tok_hbm.at[idx], rows[r % NBUF], gsem[r % NBUF])

        ahead = NBUF - 2
        pos_cp = [None, None]
        pos_cp[0] = start_pos(0)
        for cp in idx_cps:
            cp.wait()
        gcp = [None] * rounds
        wcp = [None] * rounds
        for r in range(ahead):
            gcp[r] = start_gather(r)

        for r in range(rounds):
            c, b = divmod(r, batch)
            if r + ahead < rounds:
                if r + ahead - NBUF >= 0:
                    for cp in wcp[r + ahead - NBUF]:
                        cp.wait()
                gcp[r + ahead] = start_gather(r + ahead)
            if b == 0:
                pos_cp[c % 2].wait()
                if c + 1 < chunks:
                    pos_cp[(c + 1) % 2] = start_pos(c + 1)
            gcp[r].wait()

            def add_row(i, carry, _buf=r % NBUF, _p=c % 2):
                for j in range(slices_per_row):
                    plsc.addupdate(
                        rows[_buf].at[i, pl.ds(j * LANES, LANES)],
                        pbuf[_p][i, pl.ds(j * LANES, LANES)],
                    )
                return carry

            lax.fori_loop(0, ROW_CHUNK, add_row, 0)
            wcp[r] = [
                pltpu.async_copy(
                    rows[r % NBUF],
                    out_hbm.at[pl.ds(b * seq + p0 + c * ROW_CHUNK, ROW_CHUNK)],
                    wsem[r % NBUF],
                )
            ]
        for r in range(rounds - NBUF, rounds):
            for cp in wcp[r]:
                cp.wait()

    return emb


def kernel(x, token_table, pos_table):
    batch, seq = x.shape
    vocab, d = token_table.shape
    xf = x.reshape(batch * seq).astype(jnp.int32)
    emb = _make_emb(batch, seq, vocab, d)
    out = emb(xf, token_table, pos_table)
    return out.reshape(batch, seq, d)
